# initial kernel scaffold (unmeasured)
import jax
import jax.numpy as jnp
from jax import lax
from jax.experimental import pallas as pl
from jax.experimental.pallas import tpu as pltpu

N_DEV = 16
EPS = 1e-5


def kernel(x, gamma):
    m, n_per = x.shape
    n_global = n_per * N_DEV
    g2 = gamma.reshape(1, n_per)

    def body(x_ref, g_ref, out_ref, comm_ref, send_sems, recv_sems):
        my = lax.axis_index("i")

        xv = x_ref[:, :]
        ss = jnp.sum(xv * xv, axis=1)
        comm_ref[pl.ds(my, 1), :] = ss[None, :]

        for t in range(N_DEV):
            @pl.when(my != t)
            def _(t=t):
                rdma = pltpu.make_async_remote_copy(
                    src_ref=comm_ref.at[pl.ds(my, 1)],
                    dst_ref=comm_ref.at[pl.ds(my, 1)],
                    send_sem=send_sems.at[t],
                    recv_sem=recv_sems.at[my],
                    device_id=(t,),
                    device_id_type=pl.DeviceIdType.MESH,
                )
                rdma.start()

        for s in range(N_DEV):
            @pl.when(my != s)
            def _(s=s):
                rdma = pltpu.make_async_remote_copy(
                    src_ref=comm_ref.at[pl.ds(s, 1)],
                    dst_ref=comm_ref.at[pl.ds(s, 1)],
                    send_sem=send_sems.at[s],
                    recv_sem=recv_sems.at[s],
                    device_id=(s,),
                    device_id_type=pl.DeviceIdType.MESH,
                )
                rdma.wait_recv()

        for t in range(N_DEV):
            @pl.when(my != t)
            def _(t=t):
                rdma = pltpu.make_async_remote_copy(
                    src_ref=comm_ref.at[pl.ds(my, 1)],
                    dst_ref=comm_ref.at[pl.ds(my, 1)],
                    send_sem=send_sems.at[t],
                    recv_sem=recv_sems.at[t],
                    device_id=(t,),
                    device_id_type=pl.DeviceIdType.MESH,
                )
                rdma.wait_send()

        total = jnp.sum(comm_ref[:, :], axis=0)
        inv = lax.rsqrt(total * (1.0 / n_global) + EPS)
        out_ref[:, :] = xv * inv[:, None] * g_ref[:, :]

    return pl.pallas_call(
        body,
        out_shape=jax.ShapeDtypeStruct((m, n_per), jnp.float32),
        in_specs=[
            pl.BlockSpec(memory_space=pltpu.VMEM),
            pl.BlockSpec(memory_space=pltpu.VMEM),
        ],
        out_specs=pl.BlockSpec(memory_space=pltpu.VMEM),
        scratch_shapes=[
            pltpu.VMEM((N_DEV, m), jnp.float32),
            pltpu.SemaphoreType.DMA((N_DEV,)),
            pltpu.SemaphoreType.DMA((N_DEV,)),
        ],
    )(x, g2)


# baseline (device time: 43088 ns/iter reference)
import jax
import jax.numpy as jnp
from jax import lax
from jax.experimental import pallas as pl
from jax.experimental.pallas import tpu as pltpu

N_DEV = 16
EPS = 1e-5


def kernel(x, gamma):
    m, n_per = x.shape
    n_global = n_per * N_DEV
    g2 = gamma.reshape(1, n_per)

    def body(x_ref, g_ref, out_ref, comm_ref, send_sems, recv_sems):
        my = lax.axis_index("i")

        xv = x_ref[:, :]
        ss = jnp.sum(xv * xv, axis=1)
        comm_ref[pl.ds(my, 1), :] = ss[None, :]

        for t in range(N_DEV):
            @pl.when(my != t)
            def _(t=t):
                rdma = pltpu.make_async_remote_copy(
                    src_ref=comm_ref.at[pl.ds(my, 1)],
                    dst_ref=comm_ref.at[pl.ds(my, 1)],
                    send_sem=send_sems.at[t],
                    recv_sem=recv_sems.at[my],
                    device_id=(t,),
                    device_id_type=pl.DeviceIdType.MESH,
                )
                rdma.start()

        for s in range(N_DEV):
            @pl.when(my != s)
            def _(s=s):
                rdma = pltpu.make_async_remote_copy(
                    src_ref=comm_ref.at[pl.ds(s, 1)],
                    dst_ref=comm_ref.at[pl.ds(s, 1)],
                    send_sem=send_sems.at[s],
                    recv_sem=recv_sems.at[s],
                    device_id=(s,),
                    device_id_type=pl.DeviceIdType.MESH,
                )
                rdma.wait_recv()

        for t in range(N_DEV):
            @pl.when(my != t)
            def _(t=t):
                rdma = pltpu.make_async_remote_copy(
                    src_ref=comm_ref.at[pl.ds(my, 1)],
                    dst_ref=comm_ref.at[pl.ds(my, 1)],
                    send_sem=send_sems.at[t],
                    recv_sem=recv_sems.at[t],
                    device_id=(t,),
                    device_id_type=pl.DeviceIdType.MESH,
                )
                rdma.wait_send()

        total = jnp.sum(comm_ref[:, :], axis=0)
        inv = lax.rsqrt(total * (1.0 / n_global) + EPS)
        out_ref[:, :] = xv * inv[:, None] * g_ref[:, :]

    return pl.pallas_call(
        body,
        out_shape=jax.ShapeDtypeStruct((m, n_per), jnp.float32),
        in_specs=[
            pl.BlockSpec(memory_space=pltpu.VMEM),
            pl.BlockSpec(memory_space=pltpu.VMEM),
        ],
        out_specs=pl.BlockSpec(memory_space=pltpu.VMEM),
        scratch_shapes=[
            pltpu.VMEM((N_DEV, m), jnp.float32),
            pltpu.SemaphoreType.DMA((N_DEV,)),
            pltpu.SemaphoreType.DMA((N_DEV,)),
        ],
        compiler_params=pltpu.CompilerParams(
            vmem_limit_bytes=100 * 1024 * 1024,
        ),
    )(x, g2)


# device time: 38587 ns/iter; 1.1166x vs baseline; 1.1166x over previous
import jax
import jax.numpy as jnp
from jax import lax
from jax.experimental import pallas as pl
from jax.experimental.pallas import tpu as pltpu

N_DEV = 16
EPS = 1e-5


def _allreduce_inv(x):
    m, n_per = x.shape
    n_global = n_per * N_DEV

    def body(x_ref, inv_ref, comm_ref, send_sems, recv_sems):
        my = lax.axis_index("i")

        xv = x_ref[:, :]
        ss = jnp.sum(xv * xv, axis=1)
        comm_ref[pl.ds(my, 1), :] = ss[None, :]

        for t in range(N_DEV):
            @pl.when(my != t)
            def _(t=t):
                rdma = pltpu.make_async_remote_copy(
                    src_ref=comm_ref.at[pl.ds(my, 1)],
                    dst_ref=comm_ref.at[pl.ds(my, 1)],
                    send_sem=send_sems.at[t],
                    recv_sem=recv_sems.at[my],
                    device_id=(t,),
                    device_id_type=pl.DeviceIdType.MESH,
                )
                rdma.start()

        for s in range(N_DEV):
            @pl.when(my != s)
            def _(s=s):
                rdma = pltpu.make_async_remote_copy(
                    src_ref=comm_ref.at[pl.ds(s, 1)],
                    dst_ref=comm_ref.at[pl.ds(s, 1)],
                    send_sem=send_sems.at[s],
                    recv_sem=recv_sems.at[s],
                    device_id=(s,),
                    device_id_type=pl.DeviceIdType.MESH,
                )
                rdma.wait_recv()

        for t in range(N_DEV):
            @pl.when(my != t)
            def _(t=t):
                rdma = pltpu.make_async_remote_copy(
                    src_ref=comm_ref.at[pl.ds(my, 1)],
                    dst_ref=comm_ref.at[pl.ds(my, 1)],
                    send_sem=send_sems.at[t],
                    recv_sem=recv_sems.at[t],
                    device_id=(t,),
                    device_id_type=pl.DeviceIdType.MESH,
                )
                rdma.wait_send()

        total = jnp.sum(comm_ref[:, :], axis=0)
        inv = lax.rsqrt(total * (1.0 / n_global) + EPS)
        inv_ref[:, :] = inv[:, None]

    return pl.pallas_call(
        body,
        out_shape=jax.ShapeDtypeStruct((m, 1), jnp.float32),
        in_specs=[pl.BlockSpec(memory_space=pltpu.VMEM)],
        out_specs=pl.BlockSpec(memory_space=pltpu.VMEM),
        scratch_shapes=[
            pltpu.VMEM((N_DEV, m), jnp.float32),
            pltpu.SemaphoreType.DMA((N_DEV,)),
            pltpu.SemaphoreType.DMA((N_DEV,)),
        ],
        compiler_params=pltpu.CompilerParams(
            vmem_limit_bytes=100 * 1024 * 1024,
        ),
    )(x)


def _scale(x, g2, inv):
    m, n_per = x.shape

    def body(x_ref, g_ref, inv_ref, out_ref):
        out_ref[:, :] = x_ref[:, :] * inv_ref[:, :] * g_ref[:, :]

    return pl.pallas_call(
        body,
        out_shape=jax.ShapeDtypeStruct((m, n_per), jnp.float32),
        in_specs=[
            pl.BlockSpec(memory_space=pltpu.VMEM),
            pl.BlockSpec(memory_space=pltpu.VMEM),
            pl.BlockSpec(memory_space=pltpu.VMEM),
        ],
        out_specs=pl.BlockSpec(memory_space=pltpu.VMEM),
        compiler_params=pltpu.CompilerParams(
            vmem_limit_bytes=100 * 1024 * 1024,
        ),
    )(x, g2, inv)


def kernel(x, gamma):
    m, n_per = x.shape
    g2 = gamma.reshape(1, n_per)
    inv = _allreduce_inv(x)
    return _scale(x, g2, inv)


# device time: 32286 ns/iter; 1.3346x vs baseline; 1.1952x over previous
import jax
import jax.numpy as jnp
from jax import lax
from jax.experimental import pallas as pl
from jax.experimental.pallas import tpu as pltpu

N_DEV = 16
EPS = 1e-5
N_BLK = 8


def _allreduce_inv(x):
    m, n_per = x.shape
    n_global = n_per * N_DEV
    m_blk = m // N_BLK

    def body(x_ref, inv_ref, comm_ref, send_sems, recv_sems):
        my = lax.axis_index("i")
        b = pl.program_id(0)

        @pl.when(b == 0)
        def _():
            barrier_sem = pltpu.get_barrier_semaphore()
            for t in range(N_DEV):
                @pl.when(my != t)
                def _(t=t):
                    pl.semaphore_signal(
                        barrier_sem, inc=1,
                        device_id=(t,),
                        device_id_type=pl.DeviceIdType.MESH,
                    )

        xb = x_ref[:, :]
        xb2 = xb * xb
        ones_n = jnp.ones((1, n_per), jnp.float32)
        ssb = lax.dot_general(
            ones_n, xb2,
            dimension_numbers=(((1,), (1,)), ((), ())),
            preferred_element_type=jnp.float32,
        )
        comm_ref[pl.ds(my, 1), pl.ds(b * m_blk, m_blk)] = ssb

        @pl.when(b == N_BLK - 1)
        def _():
            barrier_sem = pltpu.get_barrier_semaphore()
            pl.semaphore_wait(barrier_sem, N_DEV - 1)

            for t in range(N_DEV):
                @pl.when(my != t)
                def _(t=t):
                    rdma = pltpu.make_async_remote_copy(
                        src_ref=comm_ref.at[pl.ds(my, 1)],
                        dst_ref=comm_ref.at[pl.ds(my, 1)],
                        send_sem=send_sems.at[t],
                        recv_sem=recv_sems.at[my],
                        device_id=(t,),
                        device_id_type=pl.DeviceIdType.MESH,
                    )
                    rdma.start()

            for s in range(N_DEV):
                @pl.when(my != s)
                def _(s=s):
                    rdma = pltpu.make_async_remote_copy(
                        src_ref=comm_ref.at[pl.ds(s, 1)],
                        dst_ref=comm_ref.at[pl.ds(s, 1)],
                        send_sem=send_sems.at[s],
                        recv_sem=recv_sems.at[s],
                        device_id=(s,),
                        device_id_type=pl.DeviceIdType.MESH,
                    )
                    rdma.wait_recv()

            for t in range(N_DEV):
                @pl.when(my != t)
                def _(t=t):
                    rdma = pltpu.make_async_remote_copy(
                        src_ref=comm_ref.at[pl.ds(my, 1)],
                        dst_ref=comm_ref.at[pl.ds(my, 1)],
                        send_sem=send_sems.at[t],
                        recv_sem=recv_sems.at[t],
                        device_id=(t,),
                        device_id_type=pl.DeviceIdType.MESH,
                    )
                    rdma.wait_send()

            ones_d = jnp.ones((N_DEV, 1), jnp.float32)
            total_col = lax.dot_general(
                comm_ref[:, :], ones_d,
                dimension_numbers=(((0,), (0,)), ((), ())),
                preferred_element_type=jnp.float32,
            )
            inv_ref[:, :] = lax.rsqrt(total_col * (1.0 / n_global) + EPS)

    return pl.pallas_call(
        body,
        grid=(N_BLK,),
        out_shape=jax.ShapeDtypeStruct((m, 1), jnp.float32),
        in_specs=[
            pl.BlockSpec((m_blk, n_per), lambda b: (b, 0),
                         memory_space=pltpu.VMEM),
        ],
        out_specs=pl.BlockSpec((m, 1), lambda b: (0, 0),
                               memory_space=pltpu.VMEM),
        scratch_shapes=[
            pltpu.VMEM((N_DEV, m), jnp.float32),
            pltpu.SemaphoreType.DMA((N_DEV,)),
            pltpu.SemaphoreType.DMA((N_DEV,)),
        ],
        compiler_params=pltpu.CompilerParams(
            collective_id=0,
            dimension_semantics=("arbitrary",),
            vmem_limit_bytes=100 * 1024 * 1024,
        ),
    )(x)


def _scale(x, g2, inv):
    m, n_per = x.shape

    def body(x_ref, g_ref, inv_ref, out_ref):
        out_ref[:, :] = x_ref[:, :] * inv_ref[:, :] * g_ref[:, :]

    return pl.pallas_call(
        body,
        out_shape=jax.ShapeDtypeStruct((m, n_per), jnp.float32),
        in_specs=[
            pl.BlockSpec(memory_space=pltpu.VMEM),
            pl.BlockSpec(memory_space=pltpu.VMEM),
            pl.BlockSpec(memory_space=pltpu.VMEM),
        ],
        out_specs=pl.BlockSpec(memory_space=pltpu.VMEM),
        compiler_params=pltpu.CompilerParams(
            vmem_limit_bytes=100 * 1024 * 1024,
        ),
    )(x, g2, inv)


def kernel(x, gamma):
    m, n_per = x.shape
    g2 = gamma.reshape(1, n_per)
    inv = _allreduce_inv(x)
    return _scale(x, g2, inv)
